# Initial kernel scaffold; baseline (speedup 1.0000x reference)
import jax
import jax.numpy as jnp
from jax.experimental import pallas as pl


def kernel(time, table):
    raise NotImplementedError("write your pallas kernel here")


# speedup vs baseline: 2.2924x; 2.2924x over previous
import functools

import jax
import jax.numpy as jnp
from jax import lax
from jax.experimental import pallas as pl
from jax.experimental.pallas import tpu as pltpu
from jax.experimental.pallas import tpu_sc as plsc

NC = 2
NS = 16
NW = NC * NS
LANE = 128
K = 8
CHUNK = K * LANE


def _tec_body(idx_hbm, table_hbm, out_hbm, idx_v, rows_v, sem):
    wid = lax.axis_index("s") * NC + lax.axis_index("c")
    b_per_w = out_hbm.shape[0] // NW
    steps = b_per_w // CHUNK
    base_w = wid * b_per_w

    def step(i, carry):
        base = pl.multiple_of(base_w + i * CHUNK, CHUNK)
        pltpu.sync_copy(idx_hbm.at[pl.ds(pl.multiple_of(base // LANE, K), K)], idx_v)
        copies = [
            pltpu.async_copy(
                table_hbm.at[idx_v.at[j]],
                rows_v.at[pl.ds(j * LANE, LANE)],
                sem,
            )
            for j in range(K)
        ]
        for c in copies:
            c.wait()
        pltpu.sync_copy(rows_v, out_hbm.at[pl.ds(base, CHUNK)])
        return carry

    lax.fori_loop(0, steps, step, 0)


def kernel(time, table):
    BATCH, HIST = time.shape
    V, D = table.shape
    B = BATCH * HIST
    idx2d = time.astype(jnp.int32).reshape(B // LANE, LANE)

    mesh = plsc.VectorSubcoreMesh(core_axis_name="c", subcore_axis_name="s")
    run = pl.kernel(
        _tec_body,
        out_type=jax.ShapeDtypeStruct((B, D), jnp.float32),
        mesh=mesh,
        scratch_types=[
            pltpu.VMEM((K, LANE), jnp.int32),
            pltpu.VMEM((CHUNK, D), jnp.float32),
            pltpu.SemaphoreType.DMA,
        ],
        compiler_params=pltpu.CompilerParams(use_tc_tiling_on_sc=False),
    )
    out = run(idx2d, table)
    return out.reshape(BATCH, HIST, D)


# speedup vs baseline: 2.2996x; 1.0031x over previous
import jax
import jax.numpy as jnp
from jax import lax
from jax.experimental import pallas as pl
from jax.experimental.pallas import tpu as pltpu
from jax.experimental.pallas import tpu_sc as plsc

NC = 2
NS = 16
NW = NC * NS
LANE = 128
KG = 4
CHUNK = KG * LANE


def _tec_body(idx_hbm, table_hbm, out_hbm, idx_v, rows_v, sg0, sg1, sw0, sw1):
    D = table_hbm.shape[1]
    wid = lax.axis_index("s") * NC + lax.axis_index("c")
    steps = idx_hbm.shape[0] // NW
    rounds = steps // 2
    s0 = wid * steps
    sg = (sg0, sg1)
    sw = (sw0, sw1)

    def fire_gather(b, s):
        pltpu.sync_copy(idx_hbm.at[s], idx_v.at[b])
        for j in range(KG):
            pltpu.async_copy(
                table_hbm.at[idx_v.at[b, j]],
                rows_v.at[b].at[pl.ds(j * LANE, LANE)],
                sg[b],
            )

    def drain_gather(b):
        pltpu.make_async_copy(
            out_hbm.at[pl.ds(0, CHUNK)], rows_v.at[b], sg[b]
        ).wait()

    def fire_write(b, s):
        row0 = pl.multiple_of(s * CHUNK, CHUNK)
        pltpu.async_copy(rows_v.at[b], out_hbm.at[pl.ds(row0, CHUNK)], sw[b])

    def drain_write(b):
        pltpu.make_async_copy(
            rows_v.at[b], out_hbm.at[pl.ds(0, CHUNK)], sw[b]
        ).wait()

    fire_gather(0, s0)

    def round_(j, carry):
        s = s0 + 2 * j

        @pl.when(j >= 1)
        def _():
            drain_write(1)

        fire_gather(1, s + 1)
        drain_gather(0)
        fire_write(0, s)

        @pl.when(j < rounds - 1)
        def _():
            drain_write(0)
            fire_gather(0, s + 2)

        drain_gather(1)
        fire_write(1, s + 1)
        return carry

    lax.fori_loop(0, rounds, round_, 0)
    drain_write(0)
    drain_write(1)


def kernel(time, table):
    BATCH, HIST = time.shape
    V, D = table.shape
    B = BATCH * HIST
    idx3d = time.astype(jnp.int32).reshape(B // CHUNK, KG, LANE)

    mesh = plsc.VectorSubcoreMesh(core_axis_name="c", subcore_axis_name="s")
    run = pl.kernel(
        _tec_body,
        out_type=jax.ShapeDtypeStruct((B, D), jnp.float32),
        mesh=mesh,
        scratch_types=[
            pltpu.VMEM((2, KG, LANE), jnp.int32),
            pltpu.VMEM((2, CHUNK, D), jnp.float32),
            pltpu.SemaphoreType.DMA,
            pltpu.SemaphoreType.DMA,
            pltpu.SemaphoreType.DMA,
            pltpu.SemaphoreType.DMA,
        ],
        compiler_params=pltpu.CompilerParams(use_tc_tiling_on_sc=False),
    )
    out = run(idx3d, table)
    return out.reshape(BATCH, HIST, D)


# speedup vs baseline: 4.4010x; 1.9138x over previous
import jax
import jax.numpy as jnp
from jax import lax
from jax.experimental import pallas as pl
from jax.experimental.pallas import tpu as pltpu
from jax.experimental.pallas import tpu_sc as plsc

NC = 2
NS = 16
NW = NC * NS
H = 200
BT = 16384
V = 49
D = 64
TR = 8
SUB = 4096
TPB = SUB // 128
NQ = BT // SUB
UNITS = (H * TR) // NW


def _tec_body(idxT_hbm, tbl_hbm, out_hbm, tbl_v, idx_v, out_v, sw0, sw1):
    wid = lax.axis_index("s") * NC + lax.axis_index("c")
    pltpu.sync_copy(tbl_hbm, tbl_v)
    sw = (sw0, sw1)

    def drain(p):
        pltpu.make_async_copy(
            out_v.at[p], out_hbm.at[0, 0, pl.ds(0, TPB)], sw[p]
        ).wait()

    u0 = wid * UNITS

    def unit_body(u, carry):
        uu = u0 + u
        h = uu // TR
        tr = uu % TR
        for q in range(NQ):
            p = q % 2
            pltpu.sync_copy(idxT_hbm.at[h, pl.ds(q * SUB, SUB)], idx_v)

            @pl.when(jnp.logical_or(u > 0, q >= 2))
            def _():
                drain(p)

            @plsc.parallel_loop(0, TPB, unroll=2)
            def jbody(j):
                for bb in range(8):
                    idx16 = idx_v[pl.ds((j * 8 + bb) * 16, 16)]
                    base = idx16 * D + tr * 8
                    for ci in range(8):
                        val = plsc.load_gather(tbl_v, [base + ci])
                        out_v[p, j, ci, pl.ds(bb * 16, 16)] = val
            pltpu.async_copy(
                out_v.at[p], out_hbm.at[h, tr, pl.ds(q * TPB, TPB)], sw[p]
            )
        return carry

    lax.fori_loop(0, UNITS, unit_body, 0)
    drain(0)
    drain(1)


def kernel(time, table):
    BATCH, HIST = time.shape
    idxT = time.astype(jnp.int32).T
    tbl_flat = table.reshape(-1)

    mesh = plsc.VectorSubcoreMesh(core_axis_name="c", subcore_axis_name="s")
    run = pl.kernel(
        _tec_body,
        out_type=jax.ShapeDtypeStruct((H, TR, BT // 128, D // TR, 128), jnp.float32),
        mesh=mesh,
        scratch_types=[
            pltpu.VMEM((V * D,), jnp.float32),
            pltpu.VMEM((SUB,), jnp.int32),
            pltpu.VMEM((2, TPB, D // TR, 128), jnp.float32),
            pltpu.SemaphoreType.DMA,
            pltpu.SemaphoreType.DMA,
        ],
        compiler_params=pltpu.CompilerParams(
            use_tc_tiling_on_sc=False, needs_layout_passes=False
        ),
    )
    out5 = run(idxT, tbl_flat)
    return out5.transpose(2, 4, 0, 1, 3).reshape(BATCH, HIST, D)
